# baseline (speedup 1.0000x reference)
import functools

import numpy as np

import jax
import jax.numpy as jnp
from jax import lax
from jax.experimental import pallas as pl
from jax.experimental.pallas import tpu as pltpu
from jax.experimental.pallas import tpu_sc as plsc

N_NODES = 10000
N_EDGES = 320000
H = 64
GROUPS = 8
N_REC = 2

_NC = 2
_NS = 16
_NW = _NC * _NS
_PER_W = N_EDGES // _NW
_CH = 128
_NFULL = _PER_W // _CH
_TAIL = _PER_W - _NFULL * _CH


def _sc_mesh():
    return plsc.VectorSubcoreMesh(
        core_axis_name="c", subcore_axis_name="s",
        num_cores=_NC, num_subcores=_NS)


def _sc_gather(tab_s, tab_r, snd3, rcv3, e=N_EDGES, nf=_NFULL, tail=_TAIL):
    per_w = e // _NW
    cpw = nf + (1 if tail else 0)
    npair = (cpw + 1) // 2

    @functools.partial(
        pl.kernel,
        out_type=(jax.ShapeDtypeStruct((e, H), jnp.float32),
                  jax.ShapeDtypeStruct((e, H), jnp.float32)),
        mesh=_sc_mesh(),
        compiler_params=pltpu.CompilerParams(use_tc_tiling_on_sc=False),
        scratch_types=[
            pltpu.VMEM((cpw, _CH), jnp.int32),
            pltpu.VMEM((cpw, _CH), jnp.int32),
            pltpu.VMEM((_CH, H), jnp.float32),
            pltpu.VMEM((_CH, H), jnp.float32),
            pltpu.VMEM((_CH, H), jnp.float32),
            pltpu.VMEM((_CH, H), jnp.float32),
            pltpu.SemaphoreType.DMA,
            pltpu.SemaphoreType.DMA,
            pltpu.SemaphoreType.DMA,
            pltpu.SemaphoreType.DMA,
        ])
    def k(ts_h, tr_h, s3_h, r3_h, gs_h, gr_h,
          idxs, idxr, rs0, rr0, rs1, rr1, gsem0, gsem1, wsem0, wsem1):
        wid = lax.axis_index("s") * _NC + lax.axis_index("c")
        base = wid * per_w
        pltpu.sync_copy(s3_h.at[wid], idxs)
        pltpu.sync_copy(r3_h.at[wid], idxr)
        pltpu.async_copy(ts_h.at[idxs.at[0]], rs0, gsem0)
        pltpu.async_copy(tr_h.at[idxr.at[0]], rr0, gsem0)

        def emit(c, rs, rr, gsem, wsem):
            pltpu.make_async_copy(ts_h.at[idxs.at[c]], rs, gsem).wait()
            pltpu.make_async_copy(tr_h.at[idxr.at[c]], rr, gsem).wait()
            off = base + c * _CH

            @pl.when(c < nf)
            def _full():
                w1 = pltpu.async_copy(rs, gs_h.at[pl.ds(off, _CH)], wsem)
                w2 = pltpu.async_copy(rr, gr_h.at[pl.ds(off, _CH)], wsem)
                w1.wait()
                w2.wait()

            if tail:
                @pl.when(c == nf)
                def _tail():
                    pltpu.sync_copy(rs.at[pl.ds(0, tail)],
                                    gs_h.at[pl.ds(off, tail)])
                    pltpu.sync_copy(rr.at[pl.ds(0, tail)],
                                    gr_h.at[pl.ds(off, tail)])

        def body(kk, carry):
            c0 = 2 * kk

            @pl.when(c0 + 1 < cpw)
            def _fire1():
                pltpu.async_copy(ts_h.at[idxs.at[c0 + 1]], rs1, gsem1)
                pltpu.async_copy(tr_h.at[idxr.at[c0 + 1]], rr1, gsem1)

            emit(c0, rs0, rr0, gsem0, wsem0)

            @pl.when(c0 + 2 < cpw)
            def _fire0():
                pltpu.async_copy(ts_h.at[idxs.at[c0 + 2]], rs0, gsem0)
                pltpu.async_copy(tr_h.at[idxr.at[c0 + 2]], rr0, gsem0)

            @pl.when(c0 + 1 < cpw)
            def _emit1():
                emit(c0 + 1, rs1, rr1, gsem1, wsem1)

            return carry

        lax.fori_loop(0, npair, body, 0)

    return k(tab_s, tab_r, snd3, rcv3)


def _sc_segsum(ne, snd3, rcv3, snd, rcv, zeros_nd,
               e=N_EDGES, nf=_NFULL, tail=_TAIL):
    per_w = e // _NW
    rpt = N_NODES // _NS
    cpw = nf + (1 if tail else 0)
    npair = nf // 2

    scratch = [
        pltpu.VMEM((cpw, _CH), jnp.int32),
        pltpu.VMEM((cpw, _CH), jnp.int32),
        pltpu.VMEM((_CH, H), jnp.float32),
        pltpu.VMEM((_CH, H), jnp.float32),
        pltpu.VMEM_SHARED((N_NODES, H), jnp.float32),
        pltpu.VMEM_SHARED((N_NODES, H), jnp.float32),
        pltpu.SemaphoreType.DMA,
        pltpu.SemaphoreType.DMA,
    ]
    if tail:
        scratch += [
            pltpu.VMEM((tail,), jnp.int32),
            pltpu.VMEM((tail,), jnp.int32),
            pltpu.VMEM((tail, H), jnp.float32),
        ]

    def body_fn(ne_h, s3_h, r3_h, s_h, r_h, z_h, ps_h, pr_h,
                idxs, idxr, rows0, rows1, acc_s, acc_r, lsem0, lsem1,
                idx_st=None, idx_rt=None, rows_t=None):
        cid = lax.axis_index("c")
        sid = lax.axis_index("s")
        wid = sid * _NC + cid
        base = wid * per_w
        pltpu.sync_copy(s3_h.at[wid], idxs)
        pltpu.sync_copy(r3_h.at[wid], idxr)
        pltpu.sync_copy(z_h.at[pl.ds(sid * rpt, rpt)],
                        acc_s.at[pl.ds(sid * rpt, rpt)])
        pltpu.sync_copy(z_h.at[pl.ds(sid * rpt, rpt)],
                        acc_r.at[pl.ds(sid * rpt, rpt)])
        plsc.subcore_barrier()
        pltpu.async_copy(ne_h.at[pl.ds(base, _CH)], rows0, lsem0)

        def body(kk, carry):
            c0 = 2 * kk
            pltpu.async_copy(ne_h.at[pl.ds(base + (c0 + 1) * _CH, _CH)],
                             rows1, lsem1)
            pltpu.make_async_copy(
                ne_h.at[pl.ds(base, _CH)], rows0, lsem0).wait()
            pltpu.sync_copy(rows0, acc_s.at[idxs.at[c0]], add=True)
            pltpu.sync_copy(rows0, acc_r.at[idxr.at[c0]], add=True)

            @pl.when(c0 + 2 < nf)
            def _():
                pltpu.async_copy(
                    ne_h.at[pl.ds(base + (c0 + 2) * _CH, _CH)],
                    rows0, lsem0)

            pltpu.make_async_copy(
                ne_h.at[pl.ds(base, _CH)], rows1, lsem1).wait()
            pltpu.sync_copy(rows1, acc_s.at[idxs.at[c0 + 1]], add=True)
            pltpu.sync_copy(rows1, acc_r.at[idxr.at[c0 + 1]], add=True)
            return carry

        lax.fori_loop(0, npair, body, 0)
        if tail:
            off = base + nf * _CH
            pltpu.sync_copy(s_h.at[pl.ds(off, tail)], idx_st)
            pltpu.sync_copy(r_h.at[pl.ds(off, tail)], idx_rt)
            pltpu.sync_copy(ne_h.at[pl.ds(off, tail)], rows_t)
            pltpu.sync_copy(rows_t, acc_s.at[idx_st], add=True)
            pltpu.sync_copy(rows_t, acc_r.at[idx_rt], add=True)
        plsc.subcore_barrier()
        pltpu.sync_copy(acc_s.at[pl.ds(sid * rpt, rpt)],
                        ps_h.at[pl.ds(cid * N_NODES + sid * rpt, rpt)])
        pltpu.sync_copy(acc_r.at[pl.ds(sid * rpt, rpt)],
                        pr_h.at[pl.ds(cid * N_NODES + sid * rpt, rpt)])

    out_type = (jax.ShapeDtypeStruct((_NC * N_NODES, H), jnp.float32),
                jax.ShapeDtypeStruct((_NC * N_NODES, H), jnp.float32))
    kw = dict(
        out_type=out_type, mesh=_sc_mesh(),
        compiler_params=pltpu.CompilerParams(use_tc_tiling_on_sc=False),
        scratch_types=scratch)
    if tail:
        @functools.partial(pl.kernel, **kw)
        def k(ne_h, s3_h, r3_h, s_h, r_h, z_h, ps_h, pr_h,
              idxs, idxr, rows0, rows1, acc_s, acc_r, lsem0, lsem1,
              idx_st, idx_rt, rows_t):
            body_fn(ne_h, s3_h, r3_h, s_h, r_h, z_h, ps_h, pr_h,
                    idxs, idxr, rows0, rows1, acc_s, acc_r, lsem0, lsem1,
                    idx_st, idx_rt, rows_t)

        return k(ne, snd3, rcv3, snd, rcv, zeros_nd)

    @functools.partial(pl.kernel, **kw)
    def k0(ne_h, s3_h, r3_h, z_h, ps_h, pr_h,
           idxs, idxr, rows0, rows1, acc_s, acc_r, lsem0, lsem1):
        body_fn(ne_h, s3_h, r3_h, None, None, z_h, ps_h, pr_h,
                idxs, idxr, rows0, rows1, acc_s, acc_r, lsem0, lsem1)

    return k0(ne, snd3, rcv3, zeros_nd)


def _mm(x, w):
    return jnp.dot(x.astype(jnp.bfloat16), w.astype(jnp.bfloat16),
                   preferred_element_type=jnp.float32)


def _mm_stats(x, w):
    xh = x.astype(jnp.bfloat16)
    r1 = x - xh.astype(jnp.float32)
    xl = r1.astype(jnp.bfloat16)
    xll = (r1 - xl.astype(jnp.float32)).astype(jnp.bfloat16)
    wb = w.astype(jnp.bfloat16)
    return (jnp.dot(xh, wb, preferred_element_type=jnp.float32)
            + (jnp.dot(xl, wb, preferred_element_type=jnp.float32)
               + jnp.dot(xll, wb, preferred_element_type=jnp.float32)))


def _mlp_stage(groups, ws, adds, grow, W1, b1, sc, of, Mgn, blk,
               proj_specs=(), with_sum=False, pack_first=False,
               group_offs=None, n_rows=None):
    f32 = jnp.float32
    R = n_rows if n_rows is not None else groups[0][0].shape[0]
    if pack_first:
        R = R // 2
    grid = R // blk
    ng = len(groups)
    ws = [w if isinstance(w, (list, tuple)) else [w] for w in ws]
    flat_w = [w for wl in ws for w in wl]
    wsizes = [len(wl) for wl in ws]
    flat_x = [a for g in groups for a in g]
    gsizes = [len(g) for g in groups]
    nadds = len(adds)
    pw_flat = [w for (spec, _b) in proj_specs for (_s, w) in spec]
    pb_flat = [b for (_spec, b) in proj_specs if b is not None]
    n_proj = len(proj_specs)

    def body(*refs):
        pos = 0
        xr = refs[pos:pos + len(flat_x)]; pos += len(flat_x)
        ar = refs[pos:pos + nadds]; pos += nadds
        wr = refs[pos:pos + len(flat_w)]; pos += len(flat_w)
        growr, W1r, b1r, scr, ofr, Mr = refs[pos:pos + 6]; pos += 6
        pwr = refs[pos:pos + len(pw_flat)]; pos += len(pw_flat)
        pbr = refs[pos:pos + len(pb_flat)]; pos += len(pb_flat)
        yr = refs[pos]; pos += 1
        sumr = None
        if with_sum:
            sumr = refs[pos]; pos += 1
        projr = refs[pos:pos + n_proj]

        gv = []
        xi = 0
        for gs_ in gsizes:
            v = xr[xi][...]
            for t in range(1, gs_):
                v = v + xr[xi + t][...]
            xi += gs_
            gv.append(v)
        h = None
        wi = 0
        for j in range(ng):
            for _ in range(wsizes[j]):
                t = _mm(gv[j], wr[wi][...])
                wi += 1
                if pack_first and j == 0:
                    t = t.reshape(blk, 2 * t.shape[1])
                h = t if h is None else h + t
        for a in ar:
            h = h + a[...]
        h = jnp.maximum(h + growr[...], 0.0)
        h = jnp.maximum(_mm(h, W1r[...]) + b1r[...], 0.0)
        d = scr.shape[1]
        hh = jnp.concatenate([h, h * h], axis=1)
        mq = _mm_stats(hh, Mr[...])
        m = mq[:, :d]
        q = mq[:, d:]
        y = (h - m) * lax.rsqrt(q - m * m + 1e-5) * scr[...] + ofr[...]
        yr[...] = y
        if with_sum:
            @pl.when(pl.program_id(0) == 0)
            def _init():
                sumr[...] = jnp.zeros((1, d), f32)
            sumr[...] += jnp.sum(y, axis=0, keepdims=True)
        k = 0
        bi = 0
        for piN, (spec, b) in enumerate(proj_specs):
            accv = None
            for (src, _w) in spec:
                xv = y if src == "y" else gv[src]
                t = _mm(xv, pwr[k][...])
                k += 1
                accv = t if accv is None else accv + t
            if b is not None:
                accv = accv + pbr[bi][...]
                bi += 1
            projr[piN][...] = accv

    def rowspec(a, first=False, off=0):
        b = 2 * blk if first else blk
        return pl.BlockSpec((b, a.shape[1]), lambda i, _o=off: (i + _o, 0))

    def fullspec(a):
        return pl.BlockSpec(a.shape, lambda i: (0, 0))

    full_in = flat_w + [grow, W1, b1, sc, of, Mgn] + pw_flat + pb_flat
    in_arrays = flat_x + list(adds) + full_in
    flat_offs = []
    for gi, g in enumerate(groups):
        o = group_offs[gi] if group_offs else 0
        flat_offs += [o] * len(g)
    in_specs = ([rowspec(a, first=(pack_first and xi < gsizes[0]),
                         off=flat_offs[xi])
                 for xi, a in enumerate(flat_x)]
                + [rowspec(a) for a in adds]
                + [fullspec(a) for a in full_in])

    d = sc.shape[1]
    out_shape = [jax.ShapeDtypeStruct((R, d), f32)]
    out_specs = [pl.BlockSpec((blk, d), lambda i: (i, 0))]
    if with_sum:
        out_shape.append(jax.ShapeDtypeStruct((1, d), f32))
        out_specs.append(pl.BlockSpec((1, d), lambda i: (0, 0)))
    for (spec, _b) in proj_specs:
        dp = spec[0][1].shape[1]
        out_shape.append(jax.ShapeDtypeStruct((R, dp), f32))
        out_specs.append(pl.BlockSpec((blk, dp), lambda i: (i, 0)))

    return pl.pallas_call(
        body, grid=(grid,), in_specs=in_specs, out_specs=out_specs,
        out_shape=out_shape)(*in_arrays)


def kernel(nodes, edges, senders, receivers, globals_, params):
    p_en = params["enc_node"]
    p_ee = params["enc_edge"]
    p_eg = params["enc_glob"]
    rec = params["rec"]
    f32 = jnp.float32

    def r1(v):
        return v.reshape(1, -1)

    M1 = np.kron(np.eye(GROUPS, dtype=np.float32),
                 np.ones((H // GROUPS, H // GROUPS), dtype=np.float32)
                 / (H // GROUPS))
    Z = np.zeros_like(M1)
    Mgn = jnp.asarray(np.block([[M1, Z], [Z, M1]]))
    Mp = np.kron(np.eye(2, dtype=np.float32), M1)
    Zp = np.zeros_like(Mp)
    Mgn_p = jnp.asarray(np.block([[Mp, Zp], [Zp, Mp]]))

    def kr(w):
        return jnp.kron(jnp.eye(2, dtype=w.dtype), w)

    def t2(v):
        return jnp.tile(v.reshape(1, -1), (1, 2))

    snd = senders.astype(jnp.int32)
    rcv = receivers.astype(jnp.int32)

    def tile3(ix, per_w, nf, tail):
        cpw = nf + (1 if tail else 0)
        pad = cpw * _CH - per_w
        return jnp.pad(ix.reshape(_NW, per_w),
                       ((0, 0), (0, pad))).reshape(_NW, cpw, _CH)

    snd3 = tile3(snd, _PER_W, _NFULL, _TAIL)
    rcv3 = tile3(rcv, _PER_W, _NFULL, _TAIL)
    EA = 163840
    EB = N_EDGES - EA
    GA = (5120, 40, 0)
    GB = (4880, 38, 16)
    snd_a, snd_b = snd[:EA], snd[EA:]
    rcv_a, rcv_b = rcv[:EA], rcv[EA:]
    snd3a = tile3(snd_a, *GA)
    rcv3a = tile3(rcv_a, *GA)
    snd3b = tile3(snd_b, *GB)
    rcv3b = tile3(rcv_b, *GB)

    edges_p = edges.reshape(N_EDGES // 2, 8)
    enc_e_p = _mlp_stage([[edges_p]], [kr(p_ee["W0"])], [], t2(p_ee["b0"]),
                         kr(p_ee["W1"]), t2(p_ee["b1"]), t2(p_ee["s"]),
                         t2(p_ee["o"]), Mgn_p, 2000)[0]
    e0 = rec[0]["edge"]["W0"]
    n0 = rec[0]["node"]["W0"]
    enc_n, tab_s, tab_r = _mlp_stage(
        [[nodes]], [p_en["W0"]], [], r1(p_en["b0"]), p_en["W1"],
        r1(p_en["b1"]), r1(p_en["s"]), r1(p_en["o"]), Mgn, 1000,
        proj_specs=[([("y", e0[128:192]), ("y", e0[192:256])], None),
                    ([("y", e0[256:320]), ("y", e0[320:384])], None)])
    enc_g, gte, gtn = _mlp_stage(
        [[globals_]], [p_eg["W0"]], [], r1(p_eg["b0"]), p_eg["W1"],
        r1(p_eg["b1"]), r1(p_eg["s"]), r1(p_eg["o"]), Mgn, 1,
        proj_specs=[([("y", e0[384:448]), ("y", e0[448:512])],
                     r1(rec[0]["edge"]["b0"])),
                    ([("y", n0[256:320]), ("y", n0[320:384])],
                     r1(rec[0]["node"]["b0"]))])

    zeros_nd = jnp.zeros((N_NODES, H), f32)
    out_n, out_e_p, out_g = enc_n, enc_e_p, enc_g

    for r in range(N_REC):
        p_e = rec[r]["edge"]
        p_n = rec[r]["node"]
        p_g = rec[r]["glob"]
        W0e = p_e["W0"]
        W0n = p_n["W0"]
        W0g = p_g["W0"]

        if r == 0:
            gath_s, gath_r = _sc_gather(tab_s, tab_r, snd3, rcv3)
            gs_p = gath_s.reshape(N_EDGES // 2, 2 * H)
            gr_p = gath_r.reshape(N_EDGES // 2, 2 * H)
            new_e_p, esum_p = _mlp_stage(
                [[enc_e_p]], [[kr(W0e[0:64]), kr(W0e[64:128])]],
                [gs_p, gr_p], jnp.tile(gte, (1, 2)), kr(p_e["W1"]),
                t2(p_e["b1"]), t2(p_e["s"]), t2(p_e["o"]), Mgn_p, 2000,
                with_sum=True)
            ps, pr = _sc_segsum(new_e_p.reshape(N_EDGES, H), snd3, rcv3,
                                snd, rcv, zeros_nd)
            ps_parts = [ps[:N_NODES], ps[N_NODES:]]
            pr_parts = [pr[:N_NODES], pr[N_NODES:]]
            esums = [esum_p]
        else:
            gsA, grA = _sc_gather(tab_s, tab_r, snd3a, rcv3a,
                                  e=EA, nf=GA[1], tail=GA[2])
            gsB, grB = _sc_gather(tab_s, tab_r, snd3b, rcv3b,
                                  e=EB, nf=GB[1], tail=GB[2])
            blkh = 1280
            offb = (EA // 2) // blkh
            ewsh = [kr(W0e[0:64]), kr(W0e[64:128])]
            stage_args = (kr(p_e["W1"]), t2(p_e["b1"]), t2(p_e["s"]),
                          t2(p_e["o"]), Mgn_p, blkh)
            neA_p, esA = _mlp_stage(
                [[out_e_p], [enc_e_p]], ewsh,
                [gsA.reshape(EA // 2, 2 * H), grA.reshape(EA // 2, 2 * H)],
                jnp.tile(gte, (1, 2)), *stage_args, with_sum=True,
                group_offs=[0, 0], n_rows=EA // 2)
            neB_p, esB = _mlp_stage(
                [[out_e_p], [enc_e_p]], ewsh,
                [gsB.reshape(EB // 2, 2 * H), grB.reshape(EB // 2, 2 * H)],
                jnp.tile(gte, (1, 2)), *stage_args, with_sum=True,
                group_offs=[offb, offb], n_rows=EB // 2)
            psA, prA = _sc_segsum(neA_p.reshape(EA, H), snd3a, rcv3a,
                                  snd_a, rcv_a, zeros_nd,
                                  e=EA, nf=GA[1], tail=GA[2])
            psB, prB = _sc_segsum(neB_p.reshape(EB, H), snd3b, rcv3b,
                                  snd_b, rcv_b, zeros_nd,
                                  e=EB, nf=GB[1], tail=GB[2])
            ps_parts = [psA[:N_NODES], psA[N_NODES:],
                        psB[:N_NODES], psB[N_NODES:]]
            pr_parts = [prA[:N_NODES], prA[N_NODES:],
                        prB[:N_NODES], prB[N_NODES:]]
            esums = [esA, esB]
            new_e_p = jnp.concatenate([neA_p, neB_p], axis=0)

        if r == 0:
            ngroups = [[enc_n], ps_parts, pr_parts]
            nws = [[W0n[0:64], W0n[64:128]], W0n[128:192], W0n[192:256]]
            enc_idx = 0
        else:
            ngroups = [[out_n], [enc_n], ps_parts, pr_parts]
            nws = [W0n[0:64], W0n[64:128], W0n[128:192], W0n[192:256]]
            enc_idx = 1
        if r < N_REC - 1:
            e1 = rec[r + 1]["edge"]["W0"]
            nproj = [([("y", e1[128:192]), (enc_idx, e1[192:256])], None),
                     ([("y", e1[256:320]), (enc_idx, e1[320:384])], None)]
        else:
            nproj = []
        res = _mlp_stage(
            ngroups, nws, [], gtn, p_n["W1"], r1(p_n["b1"]), r1(p_n["s"]),
            r1(p_n["o"]), Mgn, 1000, with_sum=True, proj_specs=nproj)
        if nproj:
            new_n, nsum, tab_s, tab_r = res
        else:
            new_n, nsum = res

        esum_pair = ([es[:, :H] for es in esums]
                     + [es[:, H:] for es in esums])
        if r == 0:
            ggroups = [[nsum], esum_pair, [enc_g]]
            gws = [W0g[0:64], W0g[64:128], [W0g[128:192], W0g[192:256]]]
            encg_idx = 2
        else:
            ggroups = [[nsum], esum_pair, [out_g], [enc_g]]
            gws = [W0g[0:64], W0g[64:128], W0g[128:192], W0g[192:256]]
            encg_idx = 3
        if r < N_REC - 1:
            e1 = rec[r + 1]["edge"]["W0"]
            n1 = rec[r + 1]["node"]["W0"]
            gproj = [([("y", e1[384:448]), (encg_idx, e1[448:512])],
                      r1(rec[r + 1]["edge"]["b0"])),
                     ([("y", n1[256:320]), (encg_idx, n1[320:384])],
                      r1(rec[r + 1]["node"]["b0"]))]
        else:
            gproj = []
        res = _mlp_stage(
            ggroups, gws, [], r1(p_g["b0"]), p_g["W1"], r1(p_g["b1"]),
            r1(p_g["s"]), r1(p_g["o"]), Mgn, 1, proj_specs=gproj)
        if gproj:
            new_g, gte, gtn = res
        else:
            new_g = res[0]

        out_n, out_e_p, out_g = new_n, new_e_p, new_g

    return (out_n, out_e_p.reshape(N_EDGES, H), out_g)


# speedup vs baseline: 1.0591x; 1.0591x over previous
import functools

import numpy as np

import jax
import jax.numpy as jnp
from jax import lax
from jax.experimental import pallas as pl
from jax.experimental.pallas import tpu as pltpu
from jax.experimental.pallas import tpu_sc as plsc

N_NODES = 10000
N_EDGES = 320000
H = 64
GROUPS = 8
N_REC = 2

_NC = 2
_NS = 16
_NW = _NC * _NS
_PER_W = N_EDGES // _NW
_CH = 128
_NFULL = _PER_W // _CH
_TAIL = _PER_W - _NFULL * _CH


def _sc_mesh():
    return plsc.VectorSubcoreMesh(
        core_axis_name="c", subcore_axis_name="s",
        num_cores=_NC, num_subcores=_NS)


def _sc_gather(tab_s, tab_r, snd3, rcv3):
    cpw = _NFULL + 1

    @functools.partial(
        pl.kernel,
        out_type=(jax.ShapeDtypeStruct((N_EDGES, H), jnp.float32),
                  jax.ShapeDtypeStruct((N_EDGES, H), jnp.float32)),
        mesh=_sc_mesh(),
        compiler_params=pltpu.CompilerParams(use_tc_tiling_on_sc=False),
        scratch_types=[
            pltpu.VMEM((cpw, _CH), jnp.int32),
            pltpu.VMEM((cpw, _CH), jnp.int32),
            pltpu.VMEM((_CH, H), jnp.float32),
            pltpu.VMEM((_CH, H), jnp.float32),
            pltpu.VMEM((_CH, H), jnp.float32),
            pltpu.VMEM((_CH, H), jnp.float32),
            pltpu.SemaphoreType.DMA,
            pltpu.SemaphoreType.DMA,
            pltpu.SemaphoreType.DMA,
            pltpu.SemaphoreType.DMA,
        ])
    def k(ts_h, tr_h, s3_h, r3_h, gs_h, gr_h,
          idxs, idxr, rs0, rr0, rs1, rr1, gsem0, gsem1, wsem0, wsem1):
        wid = lax.axis_index("s") * _NC + lax.axis_index("c")
        base = wid * _PER_W
        pltpu.sync_copy(s3_h.at[wid], idxs)
        pltpu.sync_copy(r3_h.at[wid], idxr)
        pltpu.async_copy(ts_h.at[idxs.at[0]], rs0, gsem0)
        pltpu.async_copy(tr_h.at[idxr.at[0]], rr0, gsem0)

        def body(kk, carry):
            c0 = 2 * kk
            pltpu.async_copy(ts_h.at[idxs.at[c0 + 1]], rs1, gsem1)
            pltpu.async_copy(tr_h.at[idxr.at[c0 + 1]], rr1, gsem1)
            pltpu.make_async_copy(ts_h.at[idxs.at[c0]], rs0, gsem0).wait()
            pltpu.make_async_copy(tr_h.at[idxr.at[c0]], rr0, gsem0).wait()
            off0 = base + c0 * _CH
            w1 = pltpu.async_copy(rs0, gs_h.at[pl.ds(off0, _CH)], wsem0)
            w2 = pltpu.async_copy(rr0, gr_h.at[pl.ds(off0, _CH)], wsem0)
            w1.wait()
            w2.wait()
            pltpu.async_copy(ts_h.at[idxs.at[c0 + 2]], rs0, gsem0)
            pltpu.async_copy(tr_h.at[idxr.at[c0 + 2]], rr0, gsem0)
            pltpu.make_async_copy(
                ts_h.at[idxs.at[c0 + 1]], rs1, gsem1).wait()
            pltpu.make_async_copy(
                tr_h.at[idxr.at[c0 + 1]], rr1, gsem1).wait()
            off1 = off0 + _CH
            w3 = pltpu.async_copy(rs1, gs_h.at[pl.ds(off1, _CH)], wsem1)
            w4 = pltpu.async_copy(rr1, gr_h.at[pl.ds(off1, _CH)], wsem1)
            w3.wait()
            w4.wait()
            return carry

        lax.fori_loop(0, _NFULL // 2, body, 0)
        pltpu.make_async_copy(ts_h.at[idxs.at[_NFULL]], rs0, gsem0).wait()
        pltpu.make_async_copy(tr_h.at[idxr.at[_NFULL]], rr0, gsem0).wait()
        offt = base + _NFULL * _CH
        pltpu.sync_copy(rs0.at[pl.ds(0, _TAIL)],
                        gs_h.at[pl.ds(offt, _TAIL)])
        pltpu.sync_copy(rr0.at[pl.ds(0, _TAIL)],
                        gr_h.at[pl.ds(offt, _TAIL)])

    return k(tab_s, tab_r, snd3, rcv3)


def _sc_segsum(ne, snd3, rcv3, snd, rcv, zeros_nd):
    rpt = N_NODES // _NS
    cpw = _NFULL + 1

    @functools.partial(
        pl.kernel,
        out_type=(jax.ShapeDtypeStruct((_NC * N_NODES, H), jnp.float32),
                  jax.ShapeDtypeStruct((_NC * N_NODES, H), jnp.float32)),
        mesh=_sc_mesh(),
        compiler_params=pltpu.CompilerParams(use_tc_tiling_on_sc=False),
        scratch_types=[
            pltpu.VMEM((cpw, _CH), jnp.int32),
            pltpu.VMEM((cpw, _CH), jnp.int32),
            pltpu.VMEM((_CH, H), jnp.float32),
            pltpu.VMEM((_CH, H), jnp.float32),
            pltpu.VMEM((_TAIL,), jnp.int32),
            pltpu.VMEM((_TAIL,), jnp.int32),
            pltpu.VMEM((_TAIL, H), jnp.float32),
            pltpu.VMEM_SHARED((N_NODES, H), jnp.float32),
            pltpu.VMEM_SHARED((N_NODES, H), jnp.float32),
            pltpu.SemaphoreType.DMA,
            pltpu.SemaphoreType.DMA,
        ])
    def k(ne_h, s3_h, r3_h, s_h, r_h, z_h, ps_h, pr_h,
          idxs, idxr, rows0, rows1, idx_st, idx_rt, rows_t, acc_s, acc_r,
          lsem0, lsem1):
        cid = lax.axis_index("c")
        sid = lax.axis_index("s")
        wid = sid * _NC + cid
        base = wid * _PER_W
        pltpu.sync_copy(s3_h.at[wid], idxs)
        pltpu.sync_copy(r3_h.at[wid], idxr)
        pltpu.sync_copy(z_h.at[pl.ds(sid * rpt, rpt)],
                        acc_s.at[pl.ds(sid * rpt, rpt)])
        pltpu.sync_copy(z_h.at[pl.ds(sid * rpt, rpt)],
                        acc_r.at[pl.ds(sid * rpt, rpt)])
        plsc.subcore_barrier()
        pltpu.async_copy(ne_h.at[pl.ds(base, _CH)], rows0, lsem0)

        def body(kk, carry):
            c0 = 2 * kk
            pltpu.async_copy(ne_h.at[pl.ds(base + (c0 + 1) * _CH, _CH)],
                             rows1, lsem1)
            pltpu.make_async_copy(
                ne_h.at[pl.ds(base, _CH)], rows0, lsem0).wait()
            pltpu.sync_copy(rows0, acc_s.at[idxs.at[c0]], add=True)
            pltpu.sync_copy(rows0, acc_r.at[idxr.at[c0]], add=True)

            @pl.when(c0 + 2 < _NFULL)
            def _():
                pltpu.async_copy(
                    ne_h.at[pl.ds(base + (c0 + 2) * _CH, _CH)],
                    rows0, lsem0)

            pltpu.make_async_copy(
                ne_h.at[pl.ds(base, _CH)], rows1, lsem1).wait()
            pltpu.sync_copy(rows1, acc_s.at[idxs.at[c0 + 1]], add=True)
            pltpu.sync_copy(rows1, acc_r.at[idxr.at[c0 + 1]], add=True)
            return carry

        lax.fori_loop(0, _NFULL // 2, body, 0)
        off = base + _NFULL * _CH
        pltpu.sync_copy(s_h.at[pl.ds(off, _TAIL)], idx_st)
        pltpu.sync_copy(r_h.at[pl.ds(off, _TAIL)], idx_rt)
        pltpu.sync_copy(ne_h.at[pl.ds(off, _TAIL)], rows_t)
        pltpu.sync_copy(rows_t, acc_s.at[idx_st], add=True)
        pltpu.sync_copy(rows_t, acc_r.at[idx_rt], add=True)
        plsc.subcore_barrier()
        pltpu.sync_copy(acc_s.at[pl.ds(sid * rpt, rpt)],
                        ps_h.at[pl.ds(cid * N_NODES + sid * rpt, rpt)])
        pltpu.sync_copy(acc_r.at[pl.ds(sid * rpt, rpt)],
                        pr_h.at[pl.ds(cid * N_NODES + sid * rpt, rpt)])

    return k(ne, snd3, rcv3, snd, rcv, zeros_nd)


def _mm(x, w):
    return jnp.dot(x.astype(jnp.bfloat16), w.astype(jnp.bfloat16),
                   preferred_element_type=jnp.float32)


def _mm_stats(x, w):
    xh = x.astype(jnp.bfloat16)
    r1 = x - xh.astype(jnp.float32)
    xl = r1.astype(jnp.bfloat16)
    xll = (r1 - xl.astype(jnp.float32)).astype(jnp.bfloat16)
    wb = w.astype(jnp.bfloat16)
    return (jnp.dot(xh, wb, preferred_element_type=jnp.float32)
            + (jnp.dot(xl, wb, preferred_element_type=jnp.float32)
               + jnp.dot(xll, wb, preferred_element_type=jnp.float32)))


def _mlp_stage(groups, ws, adds, grow, W1, b1, sc, of, Mgn, blk,
               proj_specs=(), with_sum=False, pack_first=False):
    f32 = jnp.float32
    R = groups[0][0].shape[0]
    if pack_first:
        R = R // 2
    grid = R // blk
    ng = len(groups)
    ws = [w if isinstance(w, (list, tuple)) else [w] for w in ws]
    flat_w = [w for wl in ws for w in wl]
    wsizes = [len(wl) for wl in ws]
    flat_x = [a for g in groups for a in g]
    gsizes = [len(g) for g in groups]
    nadds = len(adds)
    pw_flat = [w for (spec, _b) in proj_specs for (_s, w) in spec]
    pb_flat = [b for (_spec, b) in proj_specs if b is not None]
    n_proj = len(proj_specs)

    def body(*refs):
        pos = 0
        xr = refs[pos:pos + len(flat_x)]; pos += len(flat_x)
        ar = refs[pos:pos + nadds]; pos += nadds
        wr = refs[pos:pos + len(flat_w)]; pos += len(flat_w)
        growr, W1r, b1r, scr, ofr, Mr = refs[pos:pos + 6]; pos += 6
        pwr = refs[pos:pos + len(pw_flat)]; pos += len(pw_flat)
        pbr = refs[pos:pos + len(pb_flat)]; pos += len(pb_flat)
        yr = refs[pos]; pos += 1
        sumr = None
        if with_sum:
            sumr = refs[pos]; pos += 1
        projr = refs[pos:pos + n_proj]

        gv = []
        xi = 0
        for gs_ in gsizes:
            v = xr[xi][...]
            for t in range(1, gs_):
                v = v + xr[xi + t][...]
            xi += gs_
            gv.append(v)
        h = None
        wi = 0
        for j in range(ng):
            for _ in range(wsizes[j]):
                t = _mm(gv[j], wr[wi][...])
                wi += 1
                if pack_first and j == 0:
                    t = t.reshape(blk, 2 * t.shape[1])
                h = t if h is None else h + t
        for a in ar:
            h = h + a[...]
        h = jnp.maximum(h + growr[...], 0.0)
        h = jnp.maximum(_mm(h, W1r[...]) + b1r[...], 0.0)
        d = scr.shape[1]
        hh = jnp.concatenate([h, h * h], axis=1)
        mq = _mm_stats(hh, Mr[...])
        m = mq[:, :d]
        q = mq[:, d:]
        y = (h - m) * lax.rsqrt(q - m * m + 1e-5) * scr[...] + ofr[...]
        yr[...] = y
        if with_sum:
            @pl.when(pl.program_id(0) == 0)
            def _init():
                sumr[...] = jnp.zeros((1, d), f32)
            sumr[...] += jnp.sum(y, axis=0, keepdims=True)
        k = 0
        bi = 0
        for piN, (spec, b) in enumerate(proj_specs):
            accv = None
            for (src, _w) in spec:
                xv = y if src == "y" else gv[src]
                t = _mm(xv, pwr[k][...])
                k += 1
                accv = t if accv is None else accv + t
            if b is not None:
                accv = accv + pbr[bi][...]
                bi += 1
            projr[piN][...] = accv

    def rowspec(a, first=False):
        b = 2 * blk if first else blk
        return pl.BlockSpec((b, a.shape[1]), lambda i: (i, 0))

    def fullspec(a):
        return pl.BlockSpec(a.shape, lambda i: (0, 0))

    full_in = flat_w + [grow, W1, b1, sc, of, Mgn] + pw_flat + pb_flat
    in_arrays = flat_x + list(adds) + full_in
    in_specs = ([rowspec(a, first=(pack_first and xi < gsizes[0]))
                 for xi, a in enumerate(flat_x)]
                + [rowspec(a) for a in adds]
                + [fullspec(a) for a in full_in])

    d = sc.shape[1]
    out_shape = [jax.ShapeDtypeStruct((R, d), f32)]
    out_specs = [pl.BlockSpec((blk, d), lambda i: (i, 0))]
    if with_sum:
        out_shape.append(jax.ShapeDtypeStruct((1, d), f32))
        out_specs.append(pl.BlockSpec((1, d), lambda i: (0, 0)))
    for (spec, _b) in proj_specs:
        dp = spec[0][1].shape[1]
        out_shape.append(jax.ShapeDtypeStruct((R, dp), f32))
        out_specs.append(pl.BlockSpec((blk, dp), lambda i: (i, 0)))

    return pl.pallas_call(
        body, grid=(grid,), in_specs=in_specs, out_specs=out_specs,
        out_shape=out_shape)(*in_arrays)


def kernel(nodes, edges, senders, receivers, globals_, params):
    p_en = params["enc_node"]
    p_ee = params["enc_edge"]
    p_eg = params["enc_glob"]
    rec = params["rec"]
    f32 = jnp.float32

    def r1(v):
        return v.reshape(1, -1)

    M1 = np.kron(np.eye(GROUPS, dtype=np.float32),
                 np.ones((H // GROUPS, H // GROUPS), dtype=np.float32)
                 / (H // GROUPS))
    Z = np.zeros_like(M1)
    Mgn = jnp.asarray(np.block([[M1, Z], [Z, M1]]))
    Mp = np.kron(np.eye(2, dtype=np.float32), M1)
    Zp = np.zeros_like(Mp)
    Mgn_p = jnp.asarray(np.block([[Mp, Zp], [Zp, Mp]]))

    def kr(w):
        return jnp.kron(jnp.eye(2, dtype=w.dtype), w)

    def t2(v):
        return jnp.tile(v.reshape(1, -1), (1, 2))

    snd = senders.astype(jnp.int32)
    rcv = receivers.astype(jnp.int32)

    def tile3(ix):
        pad = (_NFULL + 1) * _CH - _PER_W
        return jnp.pad(ix.reshape(_NW, _PER_W),
                       ((0, 0), (0, pad))).reshape(_NW, _NFULL + 1, _CH)

    snd3 = tile3(snd)
    rcv3 = tile3(rcv)

    edges_p = edges.reshape(N_EDGES // 2, 8)
    enc_e_p = _mlp_stage([[edges_p]], [kr(p_ee["W0"])], [], t2(p_ee["b0"]),
                         kr(p_ee["W1"]), t2(p_ee["b1"]), t2(p_ee["s"]),
                         t2(p_ee["o"]), Mgn_p, 2000)[0]
    e0 = rec[0]["edge"]["W0"]
    n0 = rec[0]["node"]["W0"]
    enc_n, tab_s, tab_r = _mlp_stage(
        [[nodes]], [p_en["W0"]], [], r1(p_en["b0"]), p_en["W1"],
        r1(p_en["b1"]), r1(p_en["s"]), r1(p_en["o"]), Mgn, 1000,
        proj_specs=[([("y", e0[128:192]), ("y", e0[192:256])], None),
                    ([("y", e0[256:320]), ("y", e0[320:384])], None)])
    enc_g, gte, gtn = _mlp_stage(
        [[globals_]], [p_eg["W0"]], [], r1(p_eg["b0"]), p_eg["W1"],
        r1(p_eg["b1"]), r1(p_eg["s"]), r1(p_eg["o"]), Mgn, 1,
        proj_specs=[([("y", e0[384:448]), ("y", e0[448:512])],
                     r1(rec[0]["edge"]["b0"])),
                    ([("y", n0[256:320]), ("y", n0[320:384])],
                     r1(rec[0]["node"]["b0"]))])

    zeros_nd = jnp.zeros((N_NODES, H), f32)
    out_n, out_e_p, out_g = enc_n, enc_e_p, enc_g

    for r in range(N_REC):
        p_e = rec[r]["edge"]
        p_n = rec[r]["node"]
        p_g = rec[r]["glob"]
        W0e = p_e["W0"]
        W0n = p_n["W0"]
        W0g = p_g["W0"]

        gath_s, gath_r = _sc_gather(tab_s, tab_r, snd3, rcv3)
        gs_p = gath_s.reshape(N_EDGES // 2, 2 * H)
        gr_p = gath_r.reshape(N_EDGES // 2, 2 * H)

        if r == 0:
            egroups = [[enc_e_p]]
            ews = [[kr(W0e[0:64]), kr(W0e[64:128])]]
        else:
            egroups = [[out_e_p], [enc_e_p]]
            ews = [kr(W0e[0:64]), kr(W0e[64:128])]
        new_e_p, esum_p = _mlp_stage(
            egroups, ews, [gs_p, gr_p], jnp.tile(gte, (1, 2)), kr(p_e["W1"]),
            t2(p_e["b1"]), t2(p_e["s"]), t2(p_e["o"]), Mgn_p, 2000,
            with_sum=True)

        ps, pr = _sc_segsum(new_e_p.reshape(N_EDGES, H), snd3, rcv3, snd,
                            rcv, zeros_nd)
        ps0, ps1 = ps[:N_NODES], ps[N_NODES:]
        pr0, pr1 = pr[:N_NODES], pr[N_NODES:]

        if r == 0:
            ngroups = [[enc_n], [ps0, ps1], [pr0, pr1]]
            nws = [[W0n[0:64], W0n[64:128]], W0n[128:192], W0n[192:256]]
            enc_idx = 0
        else:
            ngroups = [[out_n], [enc_n], [ps0, ps1], [pr0, pr1]]
            nws = [W0n[0:64], W0n[64:128], W0n[128:192], W0n[192:256]]
            enc_idx = 1
        if r < N_REC - 1:
            e1 = rec[r + 1]["edge"]["W0"]
            nproj = [([("y", e1[128:192]), (enc_idx, e1[192:256])], None),
                     ([("y", e1[256:320]), (enc_idx, e1[320:384])], None)]
        else:
            nproj = []
        res = _mlp_stage(
            ngroups, nws, [], gtn, p_n["W1"], r1(p_n["b1"]), r1(p_n["s"]),
            r1(p_n["o"]), Mgn, 1000, with_sum=True, proj_specs=nproj)
        if nproj:
            new_n, nsum, tab_s, tab_r = res
        else:
            new_n, nsum = res

        esum_pair = [esum_p[:, :H], esum_p[:, H:]]
        if r == 0:
            ggroups = [[nsum], esum_pair, [enc_g]]
            gws = [W0g[0:64], W0g[64:128], [W0g[128:192], W0g[192:256]]]
            encg_idx = 2
        else:
            ggroups = [[nsum], esum_pair, [out_g], [enc_g]]
            gws = [W0g[0:64], W0g[64:128], W0g[128:192], W0g[192:256]]
            encg_idx = 3
        if r < N_REC - 1:
            e1 = rec[r + 1]["edge"]["W0"]
            n1 = rec[r + 1]["node"]["W0"]
            gproj = [([("y", e1[384:448]), (encg_idx, e1[448:512])],
                      r1(rec[r + 1]["edge"]["b0"])),
                     ([("y", n1[256:320]), (encg_idx, n1[320:384])],
                      r1(rec[r + 1]["node"]["b0"]))]
        else:
            gproj = []
        res = _mlp_stage(
            ggroups, gws, [], r1(p_g["b0"]), p_g["W1"], r1(p_g["b1"]),
            r1(p_g["s"]), r1(p_g["o"]), Mgn, 1, proj_specs=gproj)
        if gproj:
            new_g, gte, gtn = res
        else:
            new_g = res[0]

        out_n, out_e_p, out_g = new_n, new_e_p, new_g

    return (out_n, out_e_p.reshape(N_EDGES, H), out_g)
